# baseline (speedup 1.0000x reference)
import jax
import jax.numpy as jnp
from jax.experimental import pallas as pl
from jax.experimental.pallas import tpu as pltpu


_PREC = jax.lax.Precision.HIGHEST


def _l2norm_cols(x):
    s = jnp.sum(x * x, axis=0, keepdims=True)
    return x * jax.lax.rsqrt(jnp.maximum(s, 1e-12))


def _null_body(sym_ref, dst_ref, w_ref, worlds_ref, c_ref):
    del sym_ref, dst_ref
    w = w_ref[0]
    x = jax.lax.dot_general(w, worlds_ref[...], (((1,), (1,)), ((), ())),
                            preferred_element_type=jnp.float32,
                            precision=_PREC)
    c_ref[0] = _l2norm_cols(x)


def _unary_body(sym_ref, xidx_ref, dst_ref, w_ref, b_ref, x_ref, cprev_ref,
                c_ref):
    del sym_ref, xidx_ref, dst_ref, cprev_ref
    w = w_ref[0]
    xs = x_ref[0]
    y = jax.lax.dot_general(w, xs, (((1,), (0,)), ((), ())),
                            preferred_element_type=jnp.float32,
                            precision=_PREC)
    c_ref[0] = _l2norm_cols(y + b_ref[0])


def _binary_body(sym_ref, lidx_ref, ridx_ref, dst_ref, w_ref, b_ref, xl_ref,
                 xr_ref, cprev_ref, c_ref):
    del sym_ref, lidx_ref, ridx_ref, dst_ref, cprev_ref
    w = w_ref[0]
    xcat = jnp.concatenate([xl_ref[0], xr_ref[0]], axis=0)
    y = jax.lax.dot_general(w, xcat, (((1,), (0,)), ((), ())),
                            preferred_element_type=jnp.float32,
                            precision=_PREC)
    c_ref[0] = _l2norm_cols(y + b_ref[0])


def _scatter_body(sidx_ref, first_ref, zero_ref, c_ref, out_ref):
    del zero_ref
    i = pl.program_id(0)
    val = c_ref[0]

    @pl.when(first_ref[i] != 0)
    def _():
        out_ref[0] = val

    @pl.when(first_ref[i] == 0)
    def _():
        out_ref[0] = out_ref[0] + val


def kernel(worlds, computed_states, null_indices, null_symbols,
           unary_indices, unary_symbols, unary_args,
           binary_indices, binary_symbols, binary_args,
           W_null, W_un, b_un, W_bin, b_bin):
    P, B, D, NW = computed_states.shape
    S = W_null.shape[0]
    n0 = null_indices.shape[0]
    n1 = unary_indices.shape[0]
    n2 = binary_indices.shape[0]
    N = n0 + n1 + n2
    i32 = jnp.int32

    nidx = null_indices.astype(i32)
    uidx = unary_indices.astype(i32)
    bidx = binary_indices.astype(i32)

    all_idx = jnp.concatenate([nidx, uidx, bidx])
    order = jnp.argsort(all_idx, stable=True)
    dst = jnp.zeros((N,), i32).at[order].set(jnp.arange(N, dtype=i32))
    sidx = jnp.take(all_idx, order)
    first = jnp.concatenate(
        [jnp.ones((1,), i32), (sidx[1:] != sidx[:-1]).astype(i32)])

    stacked = computed_states.reshape(P * B, D, NW)
    u_x = unary_args.astype(i32) * B + uidx
    l_x = binary_args[:, 0].astype(i32) * B + bidx
    r_x = binary_args[:, 1].astype(i32) * B + bidx
    b_un3 = b_un[:, :, None]
    b_bin3 = b_bin[:, :, None]

    dst_n = dst[:n0]
    dst_u = dst[n0:n0 + n1]
    dst_b = dst[n0 + n1:]

    csym = null_symbols.astype(i32)
    usym = unary_symbols.astype(i32)
    bsym = binary_symbols.astype(i32)

    cshape = jax.ShapeDtypeStruct((N, D, NW), jnp.float32)

    c1 = pl.pallas_call(
        _null_body,
        grid_spec=pltpu.PrefetchScalarGridSpec(
            num_scalar_prefetch=2,
            grid=(n0,),
            in_specs=[
                pl.BlockSpec((1, D, D), lambda i, sym, dst: (sym[i], 0, 0)),
                pl.BlockSpec((NW, D), lambda i, sym, dst: (0, 0)),
            ],
            out_specs=pl.BlockSpec((1, D, NW),
                                   lambda i, sym, dst: (dst[i], 0, 0)),
        ),
        out_shape=cshape,
        name="sat3_null",
    )(csym, dst_n, W_null, worlds)

    c2 = pl.pallas_call(
        _unary_body,
        grid_spec=pltpu.PrefetchScalarGridSpec(
            num_scalar_prefetch=3,
            grid=(n1,),
            in_specs=[
                pl.BlockSpec((1, D, D), lambda i, sym, xi, dst: (sym[i], 0, 0)),
                pl.BlockSpec((1, D, 1), lambda i, sym, xi, dst: (sym[i], 0, 0)),
                pl.BlockSpec((1, D, NW), lambda i, sym, xi, dst: (xi[i], 0, 0)),
                pl.BlockSpec(memory_space=pl.ANY),
            ],
            out_specs=pl.BlockSpec((1, D, NW),
                                   lambda i, sym, xi, dst: (dst[i], 0, 0)),
        ),
        out_shape=cshape,
        input_output_aliases={6: 0},
        name="sat3_unary",
    )(usym, u_x, dst_u, W_un, b_un3, stacked, c1)

    c3 = pl.pallas_call(
        _binary_body,
        grid_spec=pltpu.PrefetchScalarGridSpec(
            num_scalar_prefetch=4,
            grid=(n2,),
            in_specs=[
                pl.BlockSpec((1, D, 2 * D),
                             lambda i, sym, li, ri, dst: (sym[i], 0, 0)),
                pl.BlockSpec((1, D, 1),
                             lambda i, sym, li, ri, dst: (sym[i], 0, 0)),
                pl.BlockSpec((1, D, NW),
                             lambda i, sym, li, ri, dst: (li[i], 0, 0)),
                pl.BlockSpec((1, D, NW),
                             lambda i, sym, li, ri, dst: (ri[i], 0, 0)),
                pl.BlockSpec(memory_space=pl.ANY),
            ],
            out_specs=pl.BlockSpec((1, D, NW),
                                   lambda i, sym, li, ri, dst: (dst[i], 0, 0)),
        ),
        out_shape=cshape,
        input_output_aliases={8: 0},
        name="sat3_binary",
    )(bsym, l_x, r_x, dst_b, W_bin, b_bin3, stacked, stacked, c2)

    out0 = jnp.zeros((B, D, NW), jnp.float32)
    out = pl.pallas_call(
        _scatter_body,
        grid_spec=pltpu.PrefetchScalarGridSpec(
            num_scalar_prefetch=2,
            grid=(N,),
            in_specs=[
                pl.BlockSpec(memory_space=pl.ANY),
                pl.BlockSpec((1, D, NW), lambda i, sidx, first: (i, 0, 0)),
            ],
            out_specs=pl.BlockSpec((1, D, NW),
                                   lambda i, sidx, first: (sidx[i], 0, 0)),
        ),
        out_shape=jax.ShapeDtypeStruct((B, D, NW), jnp.float32),
        input_output_aliases={2: 0},
        name="sat3_scatter",
    )(sidx, first, out0, c3)

    return out


# speedup vs baseline: 3.6023x; 3.6023x over previous
import jax
import jax.numpy as jnp
from jax.experimental import pallas as pl
from jax.experimental.pallas import tpu as pltpu


_GB = 8
_SB = 64


def _l2norm_rows0(x):
    s = jnp.sum(x * x, axis=0, keepdims=True)
    return x * jax.lax.rsqrt(jnp.maximum(s, 1e-12))


def _tnull_body(w_ref, worlds_ref, t_ref):
    w = w_ref[...]
    x = jax.lax.dot_general(w, worlds_ref[...], (((2,), (1,)), ((), ())),
                            preferred_element_type=jnp.float32)
    s = jnp.sum(x * x, axis=1, keepdims=True)
    t_ref[...] = x * jax.lax.rsqrt(jnp.maximum(s, 1e-12))


def _make_main_body(P, B, D, NW, Gb):
    def body(startsN_ref, startsU_ref, startsB_ref,
             symN_ref, rowN_ref,
             symU_ref, a0U_ref, rowU_ref,
             symB_ref, a0B_ref, a1B_ref, rowB_ref,
             cs_ref, tn_ref, wun_ref, bun_ref, wbin_ref, bbin_ref,
             out_ref):
        j = pl.program_id(0)
        base = j * Gb
        out_ref[...] = jnp.zeros((Gb, D, NW), jnp.float32)

        def null_body(i, carry):
            lb = rowN_ref[i] - base
            s = symN_ref[i]
            t = tn_ref[pl.ds(s, 1)]
            out_ref[pl.ds(lb, 1)] = out_ref[pl.ds(lb, 1)] + t
            return carry

        jax.lax.fori_loop(startsN_ref[j], startsN_ref[j + 1], null_body, 0,
                          unroll=False)

        def unary_body(i, carry):
            lb = rowU_ref[i] - base
            s = symU_ref[i]
            a = a0U_ref[i]
            x = cs_ref[pl.ds(a, 1), pl.ds(lb, 1)][0, 0]
            w = wun_ref[pl.ds(s, 1)][0]
            y = jax.lax.dot_general(w, x, (((1,), (0,)), ((), ())),
                                    preferred_element_type=jnp.float32)
            b = bun_ref[pl.ds(s, 1)][0]
            y = y + jax.lax.broadcast_in_dim(b, (D, NW), (0,))
            y = _l2norm_rows0(y)
            out_ref[pl.ds(lb, 1)] = out_ref[pl.ds(lb, 1)] + y[None]
            return carry

        jax.lax.fori_loop(startsU_ref[j], startsU_ref[j + 1], unary_body, 0,
                          unroll=False)

        def binary_body(i, carry):
            lb = rowB_ref[i] - base
            s = symB_ref[i]
            a0 = a0B_ref[i]
            a1 = a1B_ref[i]
            xl = cs_ref[pl.ds(a0, 1), pl.ds(lb, 1)][0, 0]
            xr = cs_ref[pl.ds(a1, 1), pl.ds(lb, 1)][0, 0]
            w = wbin_ref[pl.ds(s, 1)][0]
            y = (jax.lax.dot_general(w[:, :D], xl, (((1,), (0,)), ((), ())),
                                     preferred_element_type=jnp.float32)
                 + jax.lax.dot_general(w[:, D:], xr, (((1,), (0,)), ((), ())),
                                       preferred_element_type=jnp.float32))
            b = bbin_ref[pl.ds(s, 1)][0]
            y = y + jax.lax.broadcast_in_dim(b, (D, NW), (0,))
            y = _l2norm_rows0(y)
            out_ref[pl.ds(lb, 1)] = out_ref[pl.ds(lb, 1)] + y[None]
            return carry

        jax.lax.fori_loop(startsB_ref[j], startsB_ref[j + 1], binary_body, 0,
                          unroll=False)

    return body


def kernel(worlds, computed_states, null_indices, null_symbols,
           unary_indices, unary_symbols, unary_args,
           binary_indices, binary_symbols, binary_args,
           W_null, W_un, b_un, W_bin, b_bin):
    P, B, D, NW = computed_states.shape
    S = W_null.shape[0]
    i32 = jnp.int32
    Gb = _GB
    nblk = B // Gb

    T_null = pl.pallas_call(
        _tnull_body,
        grid=(S // _SB,),
        in_specs=[
            pl.BlockSpec((_SB, D, D), lambda i: (i, 0, 0)),
            pl.BlockSpec((NW, D), lambda i: (0, 0)),
        ],
        out_specs=pl.BlockSpec((_SB, D, NW), lambda i: (i, 0, 0)),
        out_shape=jax.ShapeDtypeStruct((S, D, NW), jnp.float32),
        name="sat3_tnull",
    )(W_null, worlds)

    def prep(idx, *payload):
        idx = idx.astype(i32)
        order = jnp.argsort(idx)
        row = jnp.take(idx, order)
        starts = jnp.searchsorted(
            row, jnp.arange(0, B + 1, Gb, dtype=i32)).astype(i32)
        return (starts, row) + tuple(
            jnp.take(p.astype(i32), order) for p in payload)

    startsN, rowN, symN = prep(null_indices, null_symbols)
    startsU, rowU, symU, a0U = prep(unary_indices, unary_symbols, unary_args)
    startsB, rowB, symB, a0B, a1B = prep(
        binary_indices, binary_symbols, binary_args[:, 0], binary_args[:, 1])

    grid_spec = pltpu.PrefetchScalarGridSpec(
        num_scalar_prefetch=12,
        grid=(nblk,),
        in_specs=[
            pl.BlockSpec((P, Gb, D, NW),
                         lambda j, *_: (0, j, 0, 0)),
            pl.BlockSpec(memory_space=pltpu.MemorySpace.VMEM),
            pl.BlockSpec(memory_space=pltpu.MemorySpace.VMEM),
            pl.BlockSpec(memory_space=pltpu.MemorySpace.VMEM),
            pl.BlockSpec(memory_space=pltpu.MemorySpace.VMEM),
            pl.BlockSpec(memory_space=pltpu.MemorySpace.VMEM),
        ],
        out_specs=pl.BlockSpec((Gb, D, NW), lambda j, *_: (j, 0, 0)),
    )

    out = pl.pallas_call(
        _make_main_body(P, B, D, NW, Gb),
        grid_spec=grid_spec,
        out_shape=jax.ShapeDtypeStruct((B, D, NW), jnp.float32),
        name="sat3_main",
    )(startsN, startsU, startsB, symN, rowN, symU, a0U, rowU,
      symB, a0B, a1B, rowB,
      computed_states, T_null, W_un, b_un, W_bin, b_bin)

    return out


# speedup vs baseline: 3.8164x; 1.0594x over previous
import jax
import jax.numpy as jnp
from jax.experimental import pallas as pl
from jax.experimental.pallas import tpu as pltpu


_GB = 8
_SB = 64


def _l2norm_rows0(x):
    s = jnp.sum(x * x, axis=0, keepdims=True)
    return x * jax.lax.rsqrt(jnp.maximum(s, 1e-12))


def _tnull_body(w_ref, worlds_ref, t_ref):
    w = w_ref[...]
    x = jax.lax.dot_general(w, worlds_ref[...], (((2,), (1,)), ((), ())),
                            preferred_element_type=jnp.float32)
    s = jnp.sum(x * x, axis=1, keepdims=True)
    t_ref[...] = x * jax.lax.rsqrt(jnp.maximum(s, 1e-12))


def _make_main_body(P, B, D, NW, Gb):
    def body(startsN_ref, startsU_ref, startsB_ref,
             symN_ref, rowN_ref,
             symU_ref, a0U_ref, rowU_ref,
             symB_ref, a0B_ref, a1B_ref, rowB_ref,
             cs_ref, tn_ref, wun_ref, bun_ref, wbin_ref, bbin_ref,
             out_ref):
        j = pl.program_id(0)
        base = j * Gb
        out_ref[...] = jnp.zeros((Gb, D, NW), jnp.float32)

        def null_body(i, carry):
            lb = rowN_ref[i] - base
            s = symN_ref[i]
            t = tn_ref[pl.ds(s, 1)]
            out_ref[pl.ds(lb, 1)] = out_ref[pl.ds(lb, 1)] + t
            return carry

        jax.lax.fori_loop(startsN_ref[j], startsN_ref[j + 1], null_body, 0,
                          unroll=False)

        def unary_body(i, carry):
            lb = rowU_ref[i] - base
            s = symU_ref[i]
            a = a0U_ref[i]
            x = cs_ref[pl.ds(a, 1), pl.ds(lb, 1)][0, 0]
            w = wun_ref[pl.ds(s, 1)][0]
            y = jax.lax.dot_general(w, x, (((1,), (0,)), ((), ())),
                                    preferred_element_type=jnp.float32)
            b = bun_ref[pl.ds(s, 1)][0]
            y = y + jax.lax.broadcast_in_dim(b, (D, NW), (0,))
            y = _l2norm_rows0(y)
            out_ref[pl.ds(lb, 1)] = out_ref[pl.ds(lb, 1)] + y[None]
            return carry

        jax.lax.fori_loop(startsU_ref[j], startsU_ref[j + 1], unary_body, 0,
                          unroll=False)

        def binary_body(i, carry):
            lb = rowB_ref[i] - base
            s = symB_ref[i]
            a0 = a0B_ref[i]
            a1 = a1B_ref[i]
            xl = cs_ref[pl.ds(a0, 1), pl.ds(lb, 1)][0, 0]
            xr = cs_ref[pl.ds(a1, 1), pl.ds(lb, 1)][0, 0]
            w = wbin_ref[pl.ds(s, 1)][0]
            y = (jax.lax.dot_general(w[:, :D], xl, (((1,), (0,)), ((), ())),
                                     preferred_element_type=jnp.float32)
                 + jax.lax.dot_general(w[:, D:], xr, (((1,), (0,)), ((), ())),
                                       preferred_element_type=jnp.float32))
            b = bbin_ref[pl.ds(s, 1)][0]
            y = y + jax.lax.broadcast_in_dim(b, (D, NW), (0,))
            y = _l2norm_rows0(y)
            out_ref[pl.ds(lb, 1)] = out_ref[pl.ds(lb, 1)] + y[None]
            return carry

        jax.lax.fori_loop(startsB_ref[j], startsB_ref[j + 1], binary_body, 0,
                          unroll=False)

    return body


def kernel(worlds, computed_states, null_indices, null_symbols,
           unary_indices, unary_symbols, unary_args,
           binary_indices, binary_symbols, binary_args,
           W_null, W_un, b_un, W_bin, b_bin):
    P, B, D, NW = computed_states.shape
    S = W_null.shape[0]
    i32 = jnp.int32
    Gb = _GB
    nblk = B // Gb

    T_null = pl.pallas_call(
        _tnull_body,
        grid=(S // _SB,),
        in_specs=[
            pl.BlockSpec((_SB, D, D), lambda i: (i, 0, 0)),
            pl.BlockSpec((NW, D), lambda i: (0, 0)),
        ],
        out_specs=pl.BlockSpec((_SB, D, NW), lambda i: (i, 0, 0)),
        out_shape=jax.ShapeDtypeStruct((S, D, NW), jnp.float32),
        name="sat3_tnull",
    )(W_null, worlds)

    def prep_packed(idx, payloads, widths):
        key = idx.astype(i32)
        for p, w in zip(payloads, widths):
            key = (key << w) | p.astype(i32)
        key = jnp.sort(key)
        tot = sum(widths)
        row = key >> tot
        starts = jnp.searchsorted(
            row, jnp.arange(0, B + 1, Gb, dtype=i32)).astype(i32)
        outs = []
        rem = key
        for w in reversed(widths):
            outs.append(rem & ((1 << w) - 1))
            rem = rem >> w
        return (starts, row) + tuple(reversed(outs))

    sym_bits = max(1, (S - 1).bit_length())
    arg_bits = max(1, (P - 1).bit_length())
    startsN, rowN, symN = prep_packed(
        null_indices, [null_symbols], [sym_bits])
    startsU, rowU, symU, a0U = prep_packed(
        unary_indices, [unary_symbols, unary_args], [sym_bits, arg_bits])
    startsB, rowB, symB, a0B, a1B = prep_packed(
        binary_indices, [binary_symbols, binary_args[:, 0], binary_args[:, 1]],
        [sym_bits, arg_bits, arg_bits])

    grid_spec = pltpu.PrefetchScalarGridSpec(
        num_scalar_prefetch=12,
        grid=(nblk,),
        in_specs=[
            pl.BlockSpec((P, Gb, D, NW),
                         lambda j, *_: (0, j, 0, 0)),
            pl.BlockSpec(memory_space=pltpu.MemorySpace.VMEM),
            pl.BlockSpec(memory_space=pltpu.MemorySpace.VMEM),
            pl.BlockSpec(memory_space=pltpu.MemorySpace.VMEM),
            pl.BlockSpec(memory_space=pltpu.MemorySpace.VMEM),
            pl.BlockSpec(memory_space=pltpu.MemorySpace.VMEM),
        ],
        out_specs=pl.BlockSpec((Gb, D, NW), lambda j, *_: (j, 0, 0)),
    )

    out = pl.pallas_call(
        _make_main_body(P, B, D, NW, Gb),
        grid_spec=grid_spec,
        out_shape=jax.ShapeDtypeStruct((B, D, NW), jnp.float32),
        name="sat3_main",
    )(startsN, startsU, startsB, symN, rowN, symU, a0U, rowU,
      symB, a0B, a1B, rowB,
      computed_states, T_null, W_un, b_un, W_bin, b_bin)

    return out
